# Initial kernel scaffold; baseline (speedup 1.0000x reference)
import jax
import jax.numpy as jnp
from jax.experimental import pallas as pl


def kernel(x, params, edge_index, batch):
    raise NotImplementedError("write your pallas kernel here")


# speedup vs baseline: 16.1097x; 16.1097x over previous
import functools
import math

import jax
import jax.numpy as jnp
from jax import lax
from jax.experimental import pallas as pl
from jax.experimental.pallas import tpu as pltpu
from jax.experimental.pallas import tpu_sc as plsc

N = 10000
E = 160000
D_IN = 25
H = 150
NH = 5
HD = 30
HDP = 32
HP = NH * HDP
MLPD = 750
NG = 20
OUT = 6
MAX_NODES = 1000
C = 128
NCHUNK = E // C
NSUB = 16
CPW = (NCHUNK + NSUB - 1) // NSUB
MAXHC = 3
ACCW = MAXHC * HDP
BN = 1000
F32 = jnp.float32
PREC = lax.Precision.HIGHEST
HSET0 = (0, 1, 2)
HSET1 = (3, 4)


def _edge_body(q0, q1, q2, q3, q4, k0, k1, k2, k3, k4, v0, v1, v2, v3, v4,
               row_hbm, col_hbm, out_hbm,
               ridx, cidx, qr0, qr1, qr2, kr0, kr1, kr2, vr0, vr1, vr2,
               wb, shared, sem):
    qs = (q0, q1, q2, q3, q4)
    ks = (k0, k1, k2, k3, k4)
    vs = (v0, v1, v2, v3, v4)
    qrs = (qr0, qr1, qr2)
    krs = (kr0, kr1, kr2)
    vrs = (vr0, vr1, vr2)

    cid = lax.axis_index("c")
    sid = lax.axis_index("s")
    rpt = N // NSUB

    zero16 = jnp.zeros((16,), F32)

    def zrow(e, carry):
        for j in range(ACCW // 16):
            wb[e, pl.ds(16 * j, 16)] = zero16
        return carry

    lax.fori_loop(0, C, zrow, 0)

    base = sid * rpt
    nfull = rpt // C
    rem = rpt - nfull * C
    for t in range(nfull):
        pltpu.sync_copy(wb, shared.at[pl.ds(base + t * C, C)])
    if rem:
        pltpu.sync_copy(wb.at[pl.ds(0, rem)],
                        shared.at[pl.ds(base + nfull * C, rem)])
    plsc.subcore_barrier()

    def run(heads):
        nh = len(heads)

        def chunk_body(i, carry):
            c = sid + NSUB * i

            @pl.when(c < NCHUNK)
            def _():
                eb = c * C
                pltpu.sync_copy(row_hbm.at[pl.ds(eb, C)], ridx)
                pltpu.sync_copy(col_hbm.at[pl.ds(eb, C)], cidx)
                descs = []
                for j, hh in enumerate(heads):
                    descs.append(pltpu.async_copy(qs[hh].at[ridx], qrs[j], sem))
                    descs.append(pltpu.async_copy(ks[hh].at[cidx], krs[j], sem))
                    descs.append(pltpu.async_copy(vs[hh].at[cidx], vrs[j], sem))
                for d in descs:
                    d.wait()

                def edge(e, ecarry):
                    for j in range(nh):
                        a0 = qrs[j][e, pl.ds(0, 16)] * krs[j][e, pl.ds(0, 16)]
                        a1 = qrs[j][e, pl.ds(16, 16)] * krs[j][e, pl.ds(16, 16)]
                        s = jnp.sum(a0 + a1)
                        ev = jnp.exp(jnp.full((16,), s, F32))
                        wb[e, pl.ds(HDP * j, 16)] = ev * vrs[j][e, pl.ds(0, 16)]
                        wb[e, pl.ds(HDP * j + 16, 16)] = (
                            ev * vrs[j][e, pl.ds(16, 16)])
                    return ecarry

                lax.fori_loop(0, C, edge, 0)
                pltpu.sync_copy(wb, shared.at[ridx], add=True)

            return carry

        lax.fori_loop(0, CPW, chunk_body, 0)

    @pl.when(cid == 0)
    def _():
        run(HSET0)

    @pl.when(cid == 1)
    def _():
        run(HSET1)

    plsc.subcore_barrier()
    pltpu.sync_copy(shared.at[pl.ds(base, rpt)],
                    out_hbm.at[cid, pl.ds(base, rpt)])


@functools.cache
def _get_edge_call():
  return pl.kernel(
    _edge_body,
    out_type=jax.ShapeDtypeStruct((2, N, ACCW), F32),
    mesh=plsc.VectorSubcoreMesh(core_axis_name="c", subcore_axis_name="s"),
    scratch_types=[
        pltpu.VMEM((C,), jnp.int32),
        pltpu.VMEM((C,), jnp.int32),
        pltpu.VMEM((C, HDP), F32),
        pltpu.VMEM((C, HDP), F32),
        pltpu.VMEM((C, HDP), F32),
        pltpu.VMEM((C, HDP), F32),
        pltpu.VMEM((C, HDP), F32),
        pltpu.VMEM((C, HDP), F32),
        pltpu.VMEM((C, HDP), F32),
        pltpu.VMEM((C, HDP), F32),
        pltpu.VMEM((C, HDP), F32),
        pltpu.VMEM((C, ACCW), F32),
        pltpu.VMEM_SHARED((N, ACCW), F32),
        pltpu.SemaphoreType.DMA,
    ],
    compiler_params=pltpu.CompilerParams(use_tc_tiling_on_sc=False,
                                         needs_layout_passes=False),
  )


def _ln(x, g, b):
    m = jnp.mean(x, axis=-1, keepdims=True)
    v = jnp.mean((x - m) * (x - m), axis=-1, keepdims=True)
    return (x - m) * jax.lax.rsqrt(v + 1e-5) * g + b


def _embed_body(xb, bfull, bblk, win, binb, pos, ob):
    i = pl.program_id(0)
    bf = bfull[...]
    g = lax.broadcasted_iota(jnp.int32, (NG, N), 0)
    starts = jnp.sum((bf < g).astype(F32), axis=1, keepdims=True)
    myb = bblk[...]
    onz = (myb == lax.broadcasted_iota(jnp.int32, (BN, NG), 1)).astype(F32)
    sb = jnp.dot(onz, starts, preferred_element_type=F32, precision=PREC)
    rowid = i * BN + lax.broadcasted_iota(jnp.int32, (BN, 1), 0)
    pidx = jnp.clip(rowid - sb.astype(jnp.int32), 0, MAX_NODES - 1)
    P = (pidx == lax.broadcasted_iota(jnp.int32, (BN, MAX_NODES), 1)).astype(F32)
    h = jnp.dot(xb[...], win[...], preferred_element_type=F32, precision=PREC) + binb[...]
    ob[...] = h + jnp.dot(P, pos[...], preferred_element_type=F32, precision=PREC)


_embed_call = pl.pallas_call(
    _embed_body,
    grid=(N // BN,),
    in_specs=[
        pl.BlockSpec((BN, D_IN), lambda i: (i, 0)),
        pl.BlockSpec((1, N), lambda i: (0, 0)),
        pl.BlockSpec((BN, 1), lambda i: (i, 0)),
        pl.BlockSpec((D_IN, H), lambda i: (0, 0)),
        pl.BlockSpec((1, H), lambda i: (0, 0)),
        pl.BlockSpec((MAX_NODES, H), lambda i: (0, 0)),
    ],
    out_specs=pl.BlockSpec((BN, H), lambda i: (i, 0)),
    out_shape=jax.ShapeDtypeStruct((N, H), F32),
)


def _qkv_body(hb, wq, bq, wk, bk, wv, bv, *outs):
    hh = hb[...]
    q = jnp.dot(hh, wq[...], preferred_element_type=F32, precision=PREC) + bq[...]
    k = jnp.dot(hh, wk[...], preferred_element_type=F32, precision=PREC) + bk[...]
    v = jnp.dot(hh, wv[...], preferred_element_type=F32, precision=PREC) + bv[...]
    for j in range(NH):
        outs[j][...] = q[:, HDP * j:HDP * (j + 1)]
        outs[NH + j][...] = k[:, HDP * j:HDP * (j + 1)]
        outs[2 * NH + j][...] = v[:, HDP * j:HDP * (j + 1)]


_qkv_call = pl.pallas_call(
    _qkv_body,
    grid=(N // BN,),
    in_specs=[
        pl.BlockSpec((BN, H), lambda i: (i, 0)),
        pl.BlockSpec((H, HP), lambda i: (0, 0)),
        pl.BlockSpec((1, HP), lambda i: (0, 0)),
        pl.BlockSpec((H, HP), lambda i: (0, 0)),
        pl.BlockSpec((1, HP), lambda i: (0, 0)),
        pl.BlockSpec((H, HP), lambda i: (0, 0)),
        pl.BlockSpec((1, HP), lambda i: (0, 0)),
    ],
    out_specs=[pl.BlockSpec((BN, HDP), lambda i: (i, 0))] * (3 * NH),
    out_shape=[jax.ShapeDtypeStruct((N, HDP), F32)] * (3 * NH),
)


def _post_body(hb, a0, a1, wo, bo, g1, b1, w1, b1m, w2, b2m, g2, b2, ob):
    acc = jnp.concatenate([a0[...][0], a1[...][0][:, :(NH - MAXHC) * HDP]],
                          axis=1)
    den = []
    for hh in range(NH):
        r = 1.0 / (acc[:, HDP * hh + HD:HDP * hh + HD + 1] + 1e-8)
        den.append(jnp.broadcast_to(r, (BN, HDP)))
    deni = jnp.concatenate(den, axis=1)
    attn = (jnp.dot(acc * deni, wo[...], preferred_element_type=F32, precision=PREC)
            + bo[...])
    x1 = _ln(hb[...] + attn, g1[...], b1[...])
    mid = jnp.dot(x1, w1[...], preferred_element_type=F32, precision=PREC) + b1m[...]
    mid = 0.5 * mid * (1.0 + lax.erf(mid * (1.0 / math.sqrt(2.0))))
    mlp = jnp.dot(mid, w2[...], preferred_element_type=F32, precision=PREC) + b2m[...]
    ob[...] = _ln(x1 + mlp, g2[...], b2[...])


_post_call = pl.pallas_call(
    _post_body,
    grid=(N // BN,),
    in_specs=[
        pl.BlockSpec((BN, H), lambda i: (i, 0)),
        pl.BlockSpec((1, BN, ACCW), lambda i: (0, i, 0)),
        pl.BlockSpec((1, BN, ACCW), lambda i: (1, i, 0)),
        pl.BlockSpec((HP, H), lambda i: (0, 0)),
        pl.BlockSpec((1, H), lambda i: (0, 0)),
        pl.BlockSpec((1, H), lambda i: (0, 0)),
        pl.BlockSpec((1, H), lambda i: (0, 0)),
        pl.BlockSpec((H, MLPD), lambda i: (0, 0)),
        pl.BlockSpec((1, MLPD), lambda i: (0, 0)),
        pl.BlockSpec((MLPD, H), lambda i: (0, 0)),
        pl.BlockSpec((1, H), lambda i: (0, 0)),
        pl.BlockSpec((1, H), lambda i: (0, 0)),
        pl.BlockSpec((1, H), lambda i: (0, 0)),
    ],
    out_specs=pl.BlockSpec((BN, H), lambda i: (i, 0)),
    out_shape=jax.ShapeDtypeStruct((N, H), F32),
)


def _final_body(hb, bref, gf, bf_, wp1, bp1, wp2, bp2, ob):
    hn = _ln(hb[...], gf[...], bf_[...])
    bb = bref[...]
    M = (bb == lax.broadcasted_iota(jnp.int32, (NG, N), 0)).astype(F32)
    sums = jnp.dot(M, hn, preferred_element_type=F32, precision=PREC)
    counts = jnp.sum(M, axis=1, keepdims=True)
    means = sums / jnp.maximum(counts, 1.0)
    z = jnp.maximum(
        jnp.dot(means, wp1[...], preferred_element_type=F32, precision=PREC) + bp1[...], 0.0)
    ob[...] = jnp.dot(z, wp2[...], preferred_element_type=F32, precision=PREC) + bp2[...]


_final_call = pl.pallas_call(
    _final_body,
    grid=(1,),
    in_specs=[
        pl.BlockSpec((N, H), lambda i: (0, 0)),
        pl.BlockSpec((1, N), lambda i: (0, 0)),
        pl.BlockSpec((1, H), lambda i: (0, 0)),
        pl.BlockSpec((1, H), lambda i: (0, 0)),
        pl.BlockSpec((H, H // 2), lambda i: (0, 0)),
        pl.BlockSpec((1, H // 2), lambda i: (0, 0)),
        pl.BlockSpec((H // 2, OUT), lambda i: (0, 0)),
        pl.BlockSpec((1, OUT), lambda i: (0, 0)),
    ],
    out_specs=pl.BlockSpec((NG, OUT), lambda i: (0, 0)),
    out_shape=jax.ShapeDtypeStruct((NG, OUT), F32),
)


def _pad_w_cols(w):
    return jnp.pad(w.reshape(H, NH, HD), ((0, 0), (0, 0), (0, HDP - HD))
                   ).reshape(H, HP)


def _pad_b(b):
    return jnp.pad(b.reshape(NH, HD), ((0, 0), (0, HDP - HD))).reshape(1, HP)


def _pad_w_rows(w):
    return jnp.pad(w.reshape(NH, HD, H), ((0, 0), (0, HDP - HD), (0, 0))
                   ).reshape(HP, H)


def kernel(x, params, edge_index, batch):
    row = edge_index[0]
    col = edge_index[1]
    b2 = batch.reshape(1, N).astype(jnp.int32)
    bcol = batch.reshape(N, 1).astype(jnp.int32)

    h = _embed_call(x, b2, bcol, params["Win"],
                    params["bin"].reshape(1, H), params["pos"])

    isq = 1.0 / math.sqrt(HD)
    vpad_one = jnp.tile((jnp.arange(HDP) == HD).astype(F32), NH).reshape(1, HP)
    for p in params["layers"]:
        qkv = _qkv_call(
            h,
            _pad_w_cols(p["Wq"]) * isq, _pad_b(p["bq"]) * isq,
            _pad_w_cols(p["Wk"]), _pad_b(p["bk"]),
            _pad_w_cols(p["Wv"]), _pad_b(p["bv"]) + vpad_one,
        )
        parts = _get_edge_call()(*qkv, row, col)
        h = _post_call(
            h, parts, parts,
            _pad_w_rows(p["Wo"]), p["bo"].reshape(1, H),
            p["g1"].reshape(1, H), p["b1"].reshape(1, H),
            p["W1"], p["b1m"].reshape(1, MLPD),
            p["W2"], p["b2m"].reshape(1, H),
            p["g2"].reshape(1, H), p["b2"].reshape(1, H),
        )

    return _final_call(
        h, b2, params["gf"].reshape(1, H), params["bf"].reshape(1, H),
        params["Wp1"], params["bp1"].reshape(1, H // 2),
        params["Wp2"], params["bp2"].reshape(1, OUT),
    )


# speedup vs baseline: 18.6002x; 1.1546x over previous
import functools
import math

import jax
import jax.numpy as jnp
from jax import lax
from jax.experimental import pallas as pl
from jax.experimental.pallas import tpu as pltpu
from jax.experimental.pallas import tpu_sc as plsc

N = 10000
E = 160000
D_IN = 25
H = 150
NH = 5
HD = 30
HDP = 32
HP = NH * HDP
MLPD = 750
NG = 20
OUT = 6
MAX_NODES = 1000
C = 64
NCHUNK = E // C
NSUB = 16
ITER = (NCHUNK + NSUB - 1) // NSUB
NPAIR = (ITER + 1) // 2
QW = 3 * HDP
KVW = 2 * 2 * HDP + HDP + 16
ACCW = 2 * HDP + 16
BN = 1000
F32 = jnp.float32
PREC = lax.Precision.HIGHEST


def _edge_body(q0t, q1t, kv0t, kv1t, row_hbm, col_hbm, out_hbm,
               ridx0, ridx1, cidx0, cidx1, qra, qrb, kvra, kvrb,
               wb, shared, sem0, sem1):
    cid = lax.axis_index("c")
    sid = lax.axis_index("s")
    rpt = N // NSUB

    zero16 = jnp.zeros((16,), F32)

    def zrow(e, carry):
        for j in range(ACCW // 16):
            wb[e, pl.ds(16 * j, 16)] = zero16
        return carry

    lax.fori_loop(0, C, zrow, 0)

    base = sid * rpt
    nfull = rpt // C
    rem = rpt - nfull * C
    for t in range(nfull):
        pltpu.sync_copy(wb, shared.at[pl.ds(base + t * C, C)])
    if rem:
        pltpu.sync_copy(wb.at[pl.ds(0, rem)],
                        shared.at[pl.ds(base + nfull * C, rem)])
    plsc.subcore_barrier()

    ridx = (ridx0, ridx1)
    cidx = (cidx0, cidx1)
    qr = (qra, qrb)
    kvr = (kvra, kvrb)
    sems = (sem0, sem1)

    def run(qt, kvt):
        def issue(i, b):
            c = sid + NSUB * i

            @pl.when(c < NCHUNK)
            def _():
                eb = c * C
                pltpu.sync_copy(row_hbm.at[pl.ds(eb, C)], ridx[b])
                pltpu.sync_copy(col_hbm.at[pl.ds(eb, C)], cidx[b])
                pltpu.async_copy(qt.at[ridx[b]], qr[b], sems[b])
                pltpu.async_copy(kvt.at[cidx[b]], kvr[b], sems[b])

        def compute(i, b):
            c = sid + NSUB * i

            @pl.when(c < NCHUNK)
            def _():
                pltpu.make_async_copy(qt.at[ridx[b]], qr[b], sems[b]).wait()
                pltpu.make_async_copy(kvt.at[cidx[b]], kvr[b], sems[b]).wait()

                def edge(e, ecarry):
                    for j in range(2):
                        a0 = (qr[b][e, pl.ds(HDP * j, 16)]
                              * kvr[b][e, pl.ds(2 * HDP * j, 16)])
                        a1 = (qr[b][e, pl.ds(HDP * j + 16, 16)]
                              * kvr[b][e, pl.ds(2 * HDP * j + 16, 16)])
                        s = jnp.sum(a0 + a1)
                        ev = jnp.exp(jnp.full((16,), s, F32))
                        wb[e, pl.ds(HDP * j, 16)] = (
                            ev * kvr[b][e, pl.ds(2 * HDP * j + HDP, 16)])
                        wb[e, pl.ds(HDP * j + 16, 16)] = (
                            ev * kvr[b][e, pl.ds(2 * HDP * j + HDP + 16, 16)])
                    a0 = qr[b][e, pl.ds(2 * HDP, 16)] * kvr[b][e, pl.ds(4 * HDP, 16)]
                    a1 = (qr[b][e, pl.ds(2 * HDP + 16, 16)]
                          * kvr[b][e, pl.ds(4 * HDP + 16, 16)])
                    s = jnp.sum(a0 + a1)
                    ev = jnp.exp(jnp.full((16,), s, F32))
                    wb[e, pl.ds(2 * HDP, 16)] = ev * kvr[b][e, pl.ds(5 * HDP, 16)]
                    return ecarry

                lax.fori_loop(0, C, edge, 0)
                pltpu.sync_copy(wb, shared.at[ridx[b]], add=True)

        issue(0, 0)

        def pair(g, carry):
            i0 = 2 * g
            issue(i0 + 1, 1)
            compute(i0, 0)
            issue(i0 + 2, 0)
            compute(i0 + 1, 1)
            return carry

        lax.fori_loop(0, NPAIR, pair, 0)

    @pl.when(cid == 0)
    def _():
        run(q0t, kv0t)

    @pl.when(cid == 1)
    def _():
        run(q1t, kv1t)

    plsc.subcore_barrier()
    pltpu.sync_copy(shared.at[pl.ds(base, rpt)],
                    out_hbm.at[cid, pl.ds(base, rpt)])


@functools.cache
def _get_edge_call():
  return pl.kernel(
    _edge_body,
    out_type=jax.ShapeDtypeStruct((2, N, ACCW), F32),
    mesh=plsc.VectorSubcoreMesh(core_axis_name="c", subcore_axis_name="s"),
    scratch_types=[
        pltpu.VMEM((C,), jnp.int32),
        pltpu.VMEM((C,), jnp.int32),
        pltpu.VMEM((C,), jnp.int32),
        pltpu.VMEM((C,), jnp.int32),
        pltpu.VMEM((C, QW), F32),
        pltpu.VMEM((C, QW), F32),
        pltpu.VMEM((C, KVW), F32),
        pltpu.VMEM((C, KVW), F32),
        pltpu.VMEM((C, ACCW), F32),
        pltpu.VMEM_SHARED((N, ACCW), F32),
        pltpu.SemaphoreType.DMA,
        pltpu.SemaphoreType.DMA,
    ],
    compiler_params=pltpu.CompilerParams(use_tc_tiling_on_sc=False,
                                         needs_layout_passes=False),
  )


def _ln(x, g, b):
    m = jnp.mean(x, axis=-1, keepdims=True)
    v = jnp.mean((x - m) * (x - m), axis=-1, keepdims=True)
    return (x - m) * jax.lax.rsqrt(v + 1e-5) * g + b


def _embed_body(xb, bfull, bblk, win, binb, pos, ob):
    i = pl.program_id(0)
    bf = bfull[...]
    g = lax.broadcasted_iota(jnp.int32, (NG, N), 0)
    starts = jnp.sum((bf < g).astype(F32), axis=1, keepdims=True)
    myb = bblk[...]
    onz = (myb == lax.broadcasted_iota(jnp.int32, (BN, NG), 1)).astype(F32)
    sb = jnp.dot(onz, starts, preferred_element_type=F32, precision=PREC)
    rowid = i * BN + lax.broadcasted_iota(jnp.int32, (BN, 1), 0)
    pidx = jnp.clip(rowid - sb.astype(jnp.int32), 0, MAX_NODES - 1)
    P = (pidx == lax.broadcasted_iota(jnp.int32, (BN, MAX_NODES), 1)).astype(F32)
    h = jnp.dot(xb[...], win[...], preferred_element_type=F32,
                precision=PREC) + binb[...]
    ob[...] = h + jnp.dot(P, pos[...], preferred_element_type=F32,
                          precision=PREC)


_embed_call = pl.pallas_call(
    _embed_body,
    grid=(N // BN,),
    in_specs=[
        pl.BlockSpec((BN, D_IN), lambda i: (i, 0)),
        pl.BlockSpec((1, N), lambda i: (0, 0)),
        pl.BlockSpec((BN, 1), lambda i: (i, 0)),
        pl.BlockSpec((D_IN, H), lambda i: (0, 0)),
        pl.BlockSpec((1, H), lambda i: (0, 0)),
        pl.BlockSpec((MAX_NODES, H), lambda i: (0, 0)),
    ],
    out_specs=pl.BlockSpec((BN, H), lambda i: (i, 0)),
    out_shape=jax.ShapeDtypeStruct((N, H), F32),
)


def _qkv_body(hb, wq, bq, wk, bk, wv, bv, q0o, q1o, kv0o, kv1o):
    hh = hb[...]
    q = jnp.dot(hh, wq[...], preferred_element_type=F32, precision=PREC) + bq[...]
    k = jnp.dot(hh, wk[...], preferred_element_type=F32, precision=PREC) + bk[...]
    v = jnp.dot(hh, wv[...], preferred_element_type=F32, precision=PREC) + bv[...]
    q0o[...] = q[:, 0:3 * HDP]
    q1o[...] = jnp.concatenate([q[:, 3 * HDP:5 * HDP],
                                q[:, 2 * HDP:3 * HDP]], axis=1)
    kv0o[...] = jnp.concatenate([
        k[:, 0:HDP], v[:, 0:HDP],
        k[:, HDP:2 * HDP], v[:, HDP:2 * HDP],
        k[:, 2 * HDP:3 * HDP], v[:, 2 * HDP:2 * HDP + 16]], axis=1)
    kv1o[...] = jnp.concatenate([
        k[:, 3 * HDP:4 * HDP], v[:, 3 * HDP:4 * HDP],
        k[:, 4 * HDP:5 * HDP], v[:, 4 * HDP:5 * HDP],
        k[:, 2 * HDP:3 * HDP], v[:, 2 * HDP + 16:3 * HDP]], axis=1)


_qkv_call = pl.pallas_call(
    _qkv_body,
    grid=(N // BN,),
    in_specs=[
        pl.BlockSpec((BN, H), lambda i: (i, 0)),
        pl.BlockSpec((H, HP), lambda i: (0, 0)),
        pl.BlockSpec((1, HP), lambda i: (0, 0)),
        pl.BlockSpec((H, HP), lambda i: (0, 0)),
        pl.BlockSpec((1, HP), lambda i: (0, 0)),
        pl.BlockSpec((H, HP), lambda i: (0, 0)),
        pl.BlockSpec((1, HP), lambda i: (0, 0)),
    ],
    out_specs=[
        pl.BlockSpec((BN, QW), lambda i: (i, 0)),
        pl.BlockSpec((BN, QW), lambda i: (i, 0)),
        pl.BlockSpec((BN, KVW), lambda i: (i, 0)),
        pl.BlockSpec((BN, KVW), lambda i: (i, 0)),
    ],
    out_shape=[
        jax.ShapeDtypeStruct((N, QW), F32),
        jax.ShapeDtypeStruct((N, QW), F32),
        jax.ShapeDtypeStruct((N, KVW), F32),
        jax.ShapeDtypeStruct((N, KVW), F32),
    ],
)


def _post_body(hb, a0, a1, wo, bo, g1, b1, w1, b1m, w2, b2m, g2, b2, ob):
    p0 = a0[...][0]
    p1 = a1[...][0]
    acc = jnp.concatenate([p0[:, 0:2 * HDP], p0[:, 2 * HDP:],
                           p1[:, 2 * HDP:], p1[:, 0:2 * HDP]], axis=1)
    den = []
    for hh in range(NH):
        r = 1.0 / (acc[:, HDP * hh + HD:HDP * hh + HD + 1] + 1e-8)
        den.append(jnp.broadcast_to(r, (BN, HDP)))
    deni = jnp.concatenate(den, axis=1)
    attn = (jnp.dot(acc * deni, wo[...], preferred_element_type=F32,
                    precision=PREC) + bo[...])
    x1 = _ln(hb[...] + attn, g1[...], b1[...])
    mid = jnp.dot(x1, w1[...], preferred_element_type=F32,
                  precision=PREC) + b1m[...]
    mid = 0.5 * mid * (1.0 + lax.erf(mid * (1.0 / math.sqrt(2.0))))
    mlp = jnp.dot(mid, w2[...], preferred_element_type=F32,
                  precision=PREC) + b2m[...]
    ob[...] = _ln(x1 + mlp, g2[...], b2[...])


_post_call = pl.pallas_call(
    _post_body,
    grid=(N // BN,),
    in_specs=[
        pl.BlockSpec((BN, H), lambda i: (i, 0)),
        pl.BlockSpec((1, BN, ACCW), lambda i: (0, i, 0)),
        pl.BlockSpec((1, BN, ACCW), lambda i: (1, i, 0)),
        pl.BlockSpec((HP, H), lambda i: (0, 0)),
        pl.BlockSpec((1, H), lambda i: (0, 0)),
        pl.BlockSpec((1, H), lambda i: (0, 0)),
        pl.BlockSpec((1, H), lambda i: (0, 0)),
        pl.BlockSpec((H, MLPD), lambda i: (0, 0)),
        pl.BlockSpec((1, MLPD), lambda i: (0, 0)),
        pl.BlockSpec((MLPD, H), lambda i: (0, 0)),
        pl.BlockSpec((1, H), lambda i: (0, 0)),
        pl.BlockSpec((1, H), lambda i: (0, 0)),
        pl.BlockSpec((1, H), lambda i: (0, 0)),
    ],
    out_specs=pl.BlockSpec((BN, H), lambda i: (i, 0)),
    out_shape=jax.ShapeDtypeStruct((N, H), F32),
)


def _final_body(hb, bref, gf, bf_, wp1, bp1, wp2, bp2, ob):
    hn = _ln(hb[...], gf[...], bf_[...])
    bb = bref[...]
    M = (bb == lax.broadcasted_iota(jnp.int32, (NG, N), 0)).astype(F32)
    sums = jnp.dot(M, hn, preferred_element_type=F32, precision=PREC)
    counts = jnp.sum(M, axis=1, keepdims=True)
    means = sums / jnp.maximum(counts, 1.0)
    z = jnp.maximum(
        jnp.dot(means, wp1[...], preferred_element_type=F32,
                precision=PREC) + bp1[...], 0.0)
    ob[...] = jnp.dot(z, wp2[...], preferred_element_type=F32,
                      precision=PREC) + bp2[...]


_final_call = pl.pallas_call(
    _final_body,
    grid=(1,),
    in_specs=[
        pl.BlockSpec((N, H), lambda i: (0, 0)),
        pl.BlockSpec((1, N), lambda i: (0, 0)),
        pl.BlockSpec((1, H), lambda i: (0, 0)),
        pl.BlockSpec((1, H), lambda i: (0, 0)),
        pl.BlockSpec((H, H // 2), lambda i: (0, 0)),
        pl.BlockSpec((1, H // 2), lambda i: (0, 0)),
        pl.BlockSpec((H // 2, OUT), lambda i: (0, 0)),
        pl.BlockSpec((1, OUT), lambda i: (0, 0)),
    ],
    out_specs=pl.BlockSpec((NG, OUT), lambda i: (0, 0)),
    out_shape=jax.ShapeDtypeStruct((NG, OUT), F32),
)


def _pad_w_cols(w):
    return jnp.pad(w.reshape(H, NH, HD), ((0, 0), (0, 0), (0, HDP - HD))
                   ).reshape(H, HP)


def _pad_b(b):
    return jnp.pad(b.reshape(NH, HD), ((0, 0), (0, HDP - HD))).reshape(1, HP)


def _pad_w_rows(w):
    return jnp.pad(w.reshape(NH, HD, H), ((0, 0), (0, HDP - HD), (0, 0))
                   ).reshape(HP, H)


def kernel(x, params, edge_index, batch):
    row = edge_index[0]
    col = edge_index[1]
    b2 = batch.reshape(1, N).astype(jnp.int32)
    bcol = batch.reshape(N, 1).astype(jnp.int32)

    h = _embed_call(x, b2, bcol, params["Win"],
                    params["bin"].reshape(1, H), params["pos"])

    isq = 1.0 / math.sqrt(HD)
    vpad_one = jnp.tile((jnp.arange(HDP) == HD).astype(F32), NH).reshape(1, HP)
    for p in params["layers"]:
        q0t, q1t, kv0t, kv1t = _qkv_call(
            h,
            _pad_w_cols(p["Wq"]) * isq, _pad_b(p["bq"]) * isq,
            _pad_w_cols(p["Wk"]), _pad_b(p["bk"]),
            _pad_w_cols(p["Wv"]), _pad_b(p["bv"]) + vpad_one,
        )
        parts = _get_edge_call()(q0t, q1t, kv0t, kv1t, row, col)
        h = _post_call(
            h, parts, parts,
            _pad_w_rows(p["Wo"]), p["bo"].reshape(1, H),
            p["g1"].reshape(1, H), p["b1"].reshape(1, H),
            p["W1"], p["b1m"].reshape(1, MLPD),
            p["W2"], p["b2m"].reshape(1, H),
            p["g2"].reshape(1, H), p["b2"].reshape(1, H),
        )

    return _final_call(
        h, b2, params["gf"].reshape(1, H), params["bf"].reshape(1, H),
        params["Wp1"], params["bp1"].reshape(1, H // 2),
        params["Wp2"], params["bp2"].reshape(1, OUT),
    )
